# Initial kernel scaffold; baseline (speedup 1.0000x reference)
import jax
import jax.numpy as jnp
from jax.experimental import pallas as pl


def kernel(x, edge_index, W1, b1, W2, b2):
    raise NotImplementedError("write your pallas kernel here")


# speedup vs baseline: 8.9083x; 8.9083x over previous
import functools

import jax
import jax.numpy as jnp
from jax import lax
from jax.experimental import pallas as pl
from jax.experimental.pallas import tpu as pltpu
from jax.experimental.pallas import tpu_sc as plsc

N = 10000
E = 320000
D_IN = 128
D_H = 128
D_OUT = 64

NC = 2
NS = 16
NW = NC * NS

N_PAD = 10240
ROWS_PER_TILE = N_PAD // NS
E_PAD = 327680
EPW = E_PAD // NW
CHUNK = 128
NCHUNK = EPW // CHUNK

_MESH = plsc.VectorSubcoreMesh(core_axis_name="c", subcore_axis_name="s")


def _zero_fill(ref, nrows, width):
  z16 = jnp.zeros((16,), jnp.float32)

  @pl.loop(0, nrows)
  def _(i):
    for j in range(width // 16):
      ref[i, pl.ds(j * 16, 16)] = z16


def _make_sc_aggregate(d):
  params = None if d % 128 == 0 else pltpu.CompilerParams(
      use_tc_tiling_on_sc=False)

  @functools.partial(
      pl.kernel,
      out_type=jax.ShapeDtypeStruct((NC, N_PAD, d), jnp.float32),
      mesh=_MESH,
      compiler_params=params,
      scratch_types=[
          pltpu.VMEM((CHUNK,), jnp.int32),
          pltpu.VMEM((CHUNK,), jnp.int32),
          pltpu.VMEM((CHUNK, d), jnp.float32),
          pltpu.VMEM_SHARED((N_PAD, d), jnp.float32),
          pltpu.SemaphoreType.DMA,
      ],
  )
  def sc_aggregate(g_hbm, src_hbm, dst_hbm, out_hbm, src_v, dst_v, rows_v,
                   acc_sh, sem):
    c = lax.axis_index("c")
    s = lax.axis_index("s")
    w = c * NS + s

    _zero_fill(rows_v, CHUNK, d)

    @pl.loop(0, ROWS_PER_TILE // CHUNK)
    def _(i):
      pltpu.sync_copy(rows_v, acc_sh.at[pl.ds(s * ROWS_PER_TILE + i * CHUNK,
                                              CHUNK)])

    plsc.subcore_barrier()

    base = w * EPW

    @pl.loop(0, NCHUNK)
    def _(i):
      off = base + i * CHUNK
      pltpu.sync_copy(src_hbm.at[pl.ds(off, CHUNK)], src_v)
      pltpu.sync_copy(dst_hbm.at[pl.ds(off, CHUNK)], dst_v)
      pltpu.async_copy(g_hbm.at[src_v], rows_v, sem).wait()
      pltpu.sync_copy(rows_v, acc_sh.at[dst_v], add=True)

    plsc.subcore_barrier()

    pltpu.sync_copy(
        acc_sh.at[pl.ds(s * ROWS_PER_TILE, ROWS_PER_TILE)],
        out_hbm.at[c, pl.ds(s * ROWS_PER_TILE, ROWS_PER_TILE)])

  return sc_aggregate


_sc_aggregate_h = _make_sc_aggregate(D_H)
_sc_aggregate_o = _make_sc_aggregate(D_OUT)


@functools.partial(
    pl.kernel,
    out_type=jax.ShapeDtypeStruct((NC, N_PAD), jnp.float32),
    mesh=_MESH,
    scratch_types=[
        pltpu.VMEM((CHUNK,), jnp.int32),
        pltpu.VMEM((CHUNK,), jnp.float32),
        pltpu.VMEM((ROWS_PER_TILE,), jnp.float32),
        pltpu.VMEM_SHARED((N_PAD,), jnp.float32),
    ],
)
def _sc_degree(dst_hbm, out_hbm, dst_v, ones_v, zrow_v, acc_sh):
  c = lax.axis_index("c")
  s = lax.axis_index("s")
  w = c * NS + s

  one16 = jnp.ones((16,), jnp.float32)
  z16 = jnp.zeros((16,), jnp.float32)
  for j in range(CHUNK // 16):
    ones_v[pl.ds(j * 16, 16)] = one16

  @pl.loop(0, ROWS_PER_TILE // 16)
  def _(i):
    zrow_v[pl.ds(i * 16, 16)] = z16

  pltpu.sync_copy(zrow_v, acc_sh.at[pl.ds(s * ROWS_PER_TILE, ROWS_PER_TILE)])
  plsc.subcore_barrier()

  base = w * EPW

  @pl.loop(0, NCHUNK)
  def _(i):
    pltpu.sync_copy(dst_hbm.at[pl.ds(base + i * CHUNK, CHUNK)], dst_v)
    pltpu.sync_copy(ones_v, acc_sh.at[dst_v], add=True)

  plsc.subcore_barrier()

  pltpu.sync_copy(acc_sh.at[pl.ds(s * ROWS_PER_TILE, ROWS_PER_TILE)],
                  out_hbm.at[c, pl.ds(s * ROWS_PER_TILE, ROWS_PER_TILE)])


def _tc1_body(x_ref, w1_ref, degp_ref, g1_ref, dinv_ref):
  deg = degp_ref[0] + degp_ref[1] + 1.0
  dinv = lax.rsqrt(jnp.maximum(deg, 1.0))
  h = jnp.dot(x_ref[...], w1_ref[...], preferred_element_type=jnp.float32)
  g1_ref[...] = h * dinv
  dinv_ref[...] = dinv


def _tc2_body(agg_ref, g1_ref, dinv_ref, b1_ref, w2_ref, g2_ref):
  agg = agg_ref[0] + agg_ref[1]
  z1 = jnp.maximum((agg + g1_ref[...]) * dinv_ref[...] + b1_ref[...], 0.0)
  h2 = jnp.dot(z1, w2_ref[...], preferred_element_type=jnp.float32)
  g2_ref[...] = h2 * dinv_ref[...]


def _tc3_body(agg_ref, g2_ref, dinv_ref, b2_ref, out_ref):
  agg = agg_ref[0] + agg_ref[1]
  out_ref[...] = jnp.maximum((agg + g2_ref[...]) * dinv_ref[...] + b2_ref[...],
                             0.0)


_tc1 = pl.pallas_call(
    _tc1_body,
    out_shape=(jax.ShapeDtypeStruct((N_PAD, D_H), jnp.float32),
               jax.ShapeDtypeStruct((N_PAD, 1), jnp.float32)))

_tc2 = pl.pallas_call(
    _tc2_body,
    out_shape=jax.ShapeDtypeStruct((N_PAD, D_OUT), jnp.float32))

_tc3 = pl.pallas_call(
    _tc3_body,
    out_shape=jax.ShapeDtypeStruct((N_PAD, D_OUT), jnp.float32))


@jax.jit
def kernel(x, edge_index, W1, b1, W2, b2):
  src = edge_index[0].astype(jnp.int32)
  dst = edge_index[1].astype(jnp.int32)
  npad = E_PAD - E
  src_p = jnp.concatenate([src, jnp.zeros((npad,), jnp.int32)])
  dst_p = jnp.concatenate([dst, jnp.full((npad,), N, jnp.int32)])
  x_p = jnp.concatenate([x, jnp.zeros((N_PAD - N, D_IN), jnp.float32)])

  degp = _sc_degree(dst_p).reshape(NC, N_PAD, 1)
  g1, dinv = _tc1(x_p, W1, degp)
  agg1 = _sc_aggregate_h(g1, src_p, dst_p)
  g2 = _tc2(agg1, g1, dinv, b1.reshape(1, D_H), W2)
  agg2 = _sc_aggregate_o(g2, src_p, dst_p)
  z = _tc3(agg2, g2, dinv, b2.reshape(1, D_OUT))
  return z[:N]


# speedup vs baseline: 13.3050x; 1.4935x over previous
import functools

import jax
import jax.numpy as jnp
from jax import lax
from jax.experimental import pallas as pl
from jax.experimental.pallas import tpu as pltpu
from jax.experimental.pallas import tpu_sc as plsc

N = 10000
E = 320000
D_IN = 128
D_H = 128
D_OUT = 64

NC = 2
NS = 16
NW = NC * NS

N_PAD = 10240
ROWS_PER_TILE = N_PAD // NS
E_PAD = 327680
EPW = E_PAD // NW
CHUNK = 128
NCHUNK = EPW // CHUNK

_MESH = plsc.VectorSubcoreMesh(core_axis_name="c", subcore_axis_name="s")


def _zero_fill(ref, nrows, width):
  z16 = jnp.zeros((16,), jnp.float32)

  @pl.loop(0, nrows)
  def _(i):
    for j in range(width // 16):
      ref[i, pl.ds(j * 16, 16)] = z16


def _make_sc_aggregate(d):
  params = None if d % 128 == 0 else pltpu.CompilerParams(
      use_tc_tiling_on_sc=False)

  @functools.partial(
      pl.kernel,
      out_type=jax.ShapeDtypeStruct((NC, N_PAD, d), jnp.float32),
      mesh=_MESH,
      compiler_params=params,
      scratch_types=[
          pltpu.VMEM((NCHUNK // 2, CHUNK), jnp.int32),
          pltpu.VMEM((NCHUNK // 2, CHUNK), jnp.int32),
          pltpu.VMEM((CHUNK, d), jnp.float32),
          pltpu.VMEM((CHUNK, d), jnp.float32),
          pltpu.VMEM_SHARED((N_PAD, d), jnp.float32),
          pltpu.SemaphoreType.DMA,
          pltpu.SemaphoreType.DMA,
      ],
  )
  def sc_aggregate(g_hbm, src2_hbm, dst2_hbm, out_hbm, src_v, dst_v, rows0,
                   rows1, acc_sh, sem0, sem1):
    c = lax.axis_index("c")
    s = lax.axis_index("s")
    w = c * NS + s
    half = NCHUNK // 2

    _zero_fill(rows0, CHUNK, d)

    @pl.loop(0, ROWS_PER_TILE // CHUNK)
    def _(i):
      pltpu.sync_copy(rows0, acc_sh.at[pl.ds(s * ROWS_PER_TILE + i * CHUNK,
                                             CHUNK)])

    plsc.subcore_barrier()

    for h in range(2):
      row0 = w * NCHUNK + h * half
      pltpu.sync_copy(src2_hbm.at[pl.ds(row0, half)], src_v)
      pltpu.sync_copy(dst2_hbm.at[pl.ds(row0, half)], dst_v)
      pltpu.async_copy(g_hbm.at[src_v.at[0]], rows0, sem0)

      @pl.loop(0, half // 2)
      def _(i):
        j = 2 * i
        pltpu.async_copy(g_hbm.at[src_v.at[j + 1]], rows1, sem1)
        pltpu.make_async_copy(g_hbm.at[src_v.at[j]], rows0, sem0).wait()
        pltpu.sync_copy(rows0, acc_sh.at[dst_v.at[j]], add=True)

        nxt = jnp.minimum(j + 2, half - 1)

        @pl.when(i < half // 2 - 1)
        def _():
          pltpu.async_copy(g_hbm.at[src_v.at[nxt]], rows0, sem0)

        pltpu.make_async_copy(g_hbm.at[src_v.at[j + 1]], rows1, sem1).wait()
        pltpu.sync_copy(rows1, acc_sh.at[dst_v.at[j + 1]], add=True)

    plsc.subcore_barrier()

    pltpu.sync_copy(
        acc_sh.at[pl.ds(s * ROWS_PER_TILE, ROWS_PER_TILE)],
        out_hbm.at[c, pl.ds(s * ROWS_PER_TILE, ROWS_PER_TILE)])

  return sc_aggregate


_sc_aggregate_h = _make_sc_aggregate(D_H)
_sc_aggregate_o = _make_sc_aggregate(D_OUT)


@functools.partial(
    pl.kernel,
    out_type=jax.ShapeDtypeStruct((NC, N_PAD), jnp.float32),
    mesh=_MESH,
    scratch_types=[
        pltpu.VMEM((NCHUNK, CHUNK), jnp.int32),
        pltpu.VMEM((CHUNK,), jnp.float32),
        pltpu.VMEM((ROWS_PER_TILE,), jnp.float32),
        pltpu.VMEM_SHARED((N_PAD,), jnp.float32),
    ],
)
def _sc_degree(dst2_hbm, out_hbm, dst_v, ones_v, zrow_v, acc_sh):
  c = lax.axis_index("c")
  s = lax.axis_index("s")
  w = c * NS + s

  one16 = jnp.ones((16,), jnp.float32)
  z16 = jnp.zeros((16,), jnp.float32)
  for j in range(CHUNK // 16):
    ones_v[pl.ds(j * 16, 16)] = one16

  @pl.loop(0, ROWS_PER_TILE // 16)
  def _(i):
    zrow_v[pl.ds(i * 16, 16)] = z16

  pltpu.sync_copy(zrow_v, acc_sh.at[pl.ds(s * ROWS_PER_TILE, ROWS_PER_TILE)])
  pltpu.sync_copy(dst2_hbm.at[pl.ds(w * NCHUNK, NCHUNK)], dst_v)
  plsc.subcore_barrier()

  @pl.loop(0, NCHUNK)
  def _(i):
    pltpu.sync_copy(ones_v, acc_sh.at[dst_v.at[i]], add=True)

  plsc.subcore_barrier()

  pltpu.sync_copy(acc_sh.at[pl.ds(s * ROWS_PER_TILE, ROWS_PER_TILE)],
                  out_hbm.at[c, pl.ds(s * ROWS_PER_TILE, ROWS_PER_TILE)])


def _tc1_body(x_ref, w1_ref, degp_ref, g1_ref, dinv_ref):
  deg = degp_ref[0] + degp_ref[1] + 1.0
  dinv = lax.rsqrt(jnp.maximum(deg, 1.0))
  h = jnp.dot(x_ref[...], w1_ref[...], preferred_element_type=jnp.float32)
  g1_ref[...] = h * dinv
  dinv_ref[...] = dinv


def _tc2_body(agg_ref, g1_ref, dinv_ref, b1_ref, w2_ref, g2_ref):
  agg = agg_ref[0] + agg_ref[1]
  z1 = jnp.maximum((agg + g1_ref[...]) * dinv_ref[...] + b1_ref[...], 0.0)
  h2 = jnp.dot(z1, w2_ref[...], preferred_element_type=jnp.float32)
  g2_ref[...] = h2 * dinv_ref[...]


def _tc3_body(agg_ref, g2_ref, dinv_ref, b2_ref, out_ref):
  agg = agg_ref[0] + agg_ref[1]
  out_ref[...] = jnp.maximum((agg + g2_ref[...]) * dinv_ref[...] + b2_ref[...],
                             0.0)


_tc1 = pl.pallas_call(
    _tc1_body,
    out_shape=(jax.ShapeDtypeStruct((N_PAD, D_H), jnp.float32),
               jax.ShapeDtypeStruct((N_PAD, 1), jnp.float32)))

_tc2 = pl.pallas_call(
    _tc2_body,
    out_shape=jax.ShapeDtypeStruct((N_PAD, D_OUT), jnp.float32))

_tc3 = pl.pallas_call(
    _tc3_body,
    out_shape=jax.ShapeDtypeStruct((N_PAD, D_OUT), jnp.float32))


@jax.jit
def kernel(x, edge_index, W1, b1, W2, b2):
  src = edge_index[0].astype(jnp.int32)
  dst = edge_index[1].astype(jnp.int32)
  npad = E_PAD - E
  src_p = jnp.concatenate([src, jnp.zeros((npad,), jnp.int32)]).reshape(
      E_PAD // CHUNK, CHUNK)
  dst_p = jnp.concatenate([dst, jnp.full((npad,), N, jnp.int32)]).reshape(
      E_PAD // CHUNK, CHUNK)
  x_p = jnp.concatenate([x, jnp.zeros((N_PAD - N, D_IN), jnp.float32)])

  degp = _sc_degree(dst_p).reshape(NC, N_PAD, 1)
  g1, dinv = _tc1(x_p, W1, degp)
  agg1 = _sc_aggregate_h(g1, src_p, dst_p)
  g2 = _tc2(agg1, g1, dinv, b1.reshape(1, D_H), W2)
  agg2 = _sc_aggregate_o(g2, src_p, dst_p)
  z = _tc3(agg2, g2, dinv, b2.reshape(1, D_OUT))
  return z[:N]


# speedup vs baseline: 26.6294x; 2.0015x over previous
import functools

import jax
import jax.numpy as jnp
from jax import lax
from jax.experimental import pallas as pl
from jax.experimental.pallas import tpu as pltpu
from jax.experimental.pallas import tpu_sc as plsc

N = 10000
E = 320000
D_IN = 128
D_H = 128
D_OUT = 64

NC = 2
NS = 16
NW = NC * NS

N_PAD = 10240
ROWS_PER_TILE = N_PAD // NS
E_PAD = 327680
EPW = E_PAD // NW
CHUNK = 128
NCHUNK = EPW // CHUNK

_MESH = plsc.VectorSubcoreMesh(core_axis_name="c", subcore_axis_name="s")


def _zero_fill(ref, nrows, width):
  z16 = jnp.zeros((16,), jnp.float32)

  @pl.loop(0, nrows)
  def _(i):
    for j in range(width // 16):
      ref[i, pl.ds(j * 16, 16)] = z16


EPT = E_PAD // NS
NCH_T = EPT // CHUNK
SEG = 40


def _make_sc_aggregate(dh):
  params = pltpu.CompilerParams(use_tc_tiling_on_sc=False)

  @functools.partial(
      pl.kernel,
      out_type=jax.ShapeDtypeStruct((NC, N_PAD, dh), jnp.float32),
      mesh=_MESH,
      compiler_params=params,
      scratch_types=[
          pltpu.VMEM((SEG, CHUNK), jnp.int32),
          pltpu.VMEM((SEG, CHUNK), jnp.int32),
          pltpu.VMEM((CHUNK, dh), jnp.float32),
          pltpu.VMEM((CHUNK, dh), jnp.float32),
          pltpu.VMEM_SHARED((N_PAD, dh), jnp.float32),
          pltpu.VMEM_SHARED((N_PAD, dh), jnp.float32),
          pltpu.SemaphoreType.DMA,
          pltpu.SemaphoreType.DMA,
      ],
  )
  def sc_aggregate(g_hbm, src2_hbm, dst2_hbm, out_hbm, src_v, dst_v, rows0,
                   rows1, table_sh, acc_sh, sem0, sem1):
    c = lax.axis_index("c")
    s = lax.axis_index("s")
    rslice = pl.ds(s * ROWS_PER_TILE, ROWS_PER_TILE)

    pltpu.sync_copy(g_hbm.at[c, rslice], table_sh.at[rslice])
    _zero_fill(rows0, CHUNK, dh)

    @pl.loop(0, ROWS_PER_TILE // CHUNK)
    def _(i):
      pltpu.sync_copy(rows0, acc_sh.at[pl.ds(s * ROWS_PER_TILE + i * CHUNK,
                                             CHUNK)])

    plsc.subcore_barrier()

    for seg in range(NCH_T // SEG):
      row0 = s * NCH_T + seg * SEG
      pltpu.sync_copy(src2_hbm.at[pl.ds(row0, SEG)], src_v)
      pltpu.sync_copy(dst2_hbm.at[pl.ds(row0, SEG)], dst_v)
      pltpu.async_copy(table_sh.at[src_v.at[0]], rows0, sem0)

      @pl.loop(0, SEG // 2)
      def _(i):
        j = 2 * i
        pltpu.async_copy(table_sh.at[src_v.at[j + 1]], rows1, sem1)
        pltpu.make_async_copy(table_sh.at[src_v.at[j]], rows0, sem0).wait()
        pltpu.sync_copy(rows0, acc_sh.at[dst_v.at[j]], add=True)

        nxt = jnp.minimum(j + 2, SEG - 1)

        @pl.when(i < SEG // 2 - 1)
        def _():
          pltpu.async_copy(table_sh.at[src_v.at[nxt]], rows0, sem0)

        pltpu.make_async_copy(table_sh.at[src_v.at[j + 1]], rows1, sem1).wait()
        pltpu.sync_copy(rows1, acc_sh.at[dst_v.at[j + 1]], add=True)

    plsc.subcore_barrier()

    pltpu.sync_copy(acc_sh.at[rslice], out_hbm.at[c, rslice])

  return sc_aggregate


_sc_aggregate_h = _make_sc_aggregate(D_H // NC)
_sc_aggregate_o = _make_sc_aggregate(D_OUT // NC)


@functools.partial(
    pl.kernel,
    out_type=jax.ShapeDtypeStruct((NC, N_PAD), jnp.float32),
    mesh=_MESH,
    scratch_types=[
        pltpu.VMEM((NCHUNK, CHUNK), jnp.int32),
        pltpu.VMEM((CHUNK,), jnp.float32),
        pltpu.VMEM((ROWS_PER_TILE,), jnp.float32),
        pltpu.VMEM_SHARED((N_PAD,), jnp.float32),
    ],
)
def _sc_degree(dst2_hbm, out_hbm, dst_v, ones_v, zrow_v, acc_sh):
  c = lax.axis_index("c")
  s = lax.axis_index("s")
  w = c * NS + s

  one16 = jnp.ones((16,), jnp.float32)
  z16 = jnp.zeros((16,), jnp.float32)
  for j in range(CHUNK // 16):
    ones_v[pl.ds(j * 16, 16)] = one16

  @pl.loop(0, ROWS_PER_TILE // 16)
  def _(i):
    zrow_v[pl.ds(i * 16, 16)] = z16

  pltpu.sync_copy(zrow_v, acc_sh.at[pl.ds(s * ROWS_PER_TILE, ROWS_PER_TILE)])
  pltpu.sync_copy(dst2_hbm.at[pl.ds(w * NCHUNK, NCHUNK)], dst_v)
  plsc.subcore_barrier()

  @pl.loop(0, NCHUNK)
  def _(i):
    pltpu.sync_copy(ones_v, acc_sh.at[dst_v.at[i]], add=True)

  plsc.subcore_barrier()

  pltpu.sync_copy(acc_sh.at[pl.ds(s * ROWS_PER_TILE, ROWS_PER_TILE)],
                  out_hbm.at[c, pl.ds(s * ROWS_PER_TILE, ROWS_PER_TILE)])


def _split(h):
  return jnp.stack([h[:, :h.shape[1] // NC], h[:, h.shape[1] // NC:]])


def _cat(ref):
  return jnp.concatenate([ref[0], ref[1]], axis=-1)


def _tc1_body(x_ref, w1_ref, degp_ref, g1_ref, dinv_ref):
  deg = degp_ref[0] + degp_ref[1] + 1.0
  dinv = lax.rsqrt(jnp.maximum(deg, 1.0))
  h = jnp.dot(x_ref[...], w1_ref[...], preferred_element_type=jnp.float32)
  g1_ref[...] = _split(h * dinv)
  dinv_ref[...] = dinv


def _tc2_body(agg_ref, g1_ref, dinv_ref, b1_ref, w2_ref, g2_ref):
  agg = _cat(agg_ref) + _cat(g1_ref)
  z1 = jnp.maximum(agg * dinv_ref[...] + b1_ref[...], 0.0)
  h2 = jnp.dot(z1, w2_ref[...], preferred_element_type=jnp.float32)
  g2_ref[...] = _split(h2 * dinv_ref[...])


def _tc3_body(agg_ref, g2_ref, dinv_ref, b2_ref, out_ref):
  agg = _cat(agg_ref) + _cat(g2_ref)
  out_ref[...] = jnp.maximum(agg * dinv_ref[...] + b2_ref[...], 0.0)


_tc1 = pl.pallas_call(
    _tc1_body,
    out_shape=(jax.ShapeDtypeStruct((NC, N_PAD, D_H // NC), jnp.float32),
               jax.ShapeDtypeStruct((N_PAD, 1), jnp.float32)))

_tc2 = pl.pallas_call(
    _tc2_body,
    out_shape=jax.ShapeDtypeStruct((NC, N_PAD, D_OUT // NC), jnp.float32))

_tc3 = pl.pallas_call(
    _tc3_body,
    out_shape=jax.ShapeDtypeStruct((N_PAD, D_OUT), jnp.float32))


@jax.jit
def kernel(x, edge_index, W1, b1, W2, b2):
  src = edge_index[0].astype(jnp.int32)
  dst = edge_index[1].astype(jnp.int32)
  npad = E_PAD - E
  src_p = jnp.concatenate([src, jnp.zeros((npad,), jnp.int32)]).reshape(
      E_PAD // CHUNK, CHUNK)
  dst_p = jnp.concatenate([dst, jnp.full((npad,), N, jnp.int32)]).reshape(
      E_PAD // CHUNK, CHUNK)
  x_p = jnp.concatenate([x, jnp.zeros((N_PAD - N, D_IN), jnp.float32)])

  degp = _sc_degree(dst_p).reshape(NC, N_PAD, 1)
  g1, dinv = _tc1(x_p, W1, degp)
  agg1 = _sc_aggregate_h(g1, src_p, dst_p)
  g2 = _tc2(agg1, g1, dinv, b1.reshape(1, D_H), W2)
  agg2 = _sc_aggregate_o(g2, src_p, dst_p)
  z = _tc3(agg2, g2, dinv, b2.reshape(1, D_OUT))
  return z[:N]


# speedup vs baseline: 29.4344x; 1.1053x over previous
import functools

import jax
import jax.numpy as jnp
from jax import lax
from jax.experimental import pallas as pl
from jax.experimental.pallas import tpu as pltpu
from jax.experimental.pallas import tpu_sc as plsc

N = 10000
E = 320000
D_IN = 128
D_H = 128
D_OUT = 64

NC = 2
NS = 16
NW = NC * NS

N_PAD = 10240
ROWS_PER_TILE = N_PAD // NS
E_PAD = 327680
EPW = E_PAD // NW
CHUNK = 128
NCHUNK = EPW // CHUNK

_MESH = plsc.VectorSubcoreMesh(core_axis_name="c", subcore_axis_name="s")


def _zero_fill(ref, nrows, width):
  z16 = jnp.zeros((16,), jnp.float32)

  @pl.loop(0, nrows)
  def _(i):
    for j in range(width // 16):
      ref[i, pl.ds(j * 16, 16)] = z16


EPT = E_PAD // NS
NCH_T = EPT // CHUNK
SEG = 32
NBUF = 4


def _make_sc_aggregate(dh):
  params = pltpu.CompilerParams(use_tc_tiling_on_sc=False)

  @functools.partial(
      pl.kernel,
      out_type=jax.ShapeDtypeStruct((NC, N_PAD, dh), jnp.float32),
      mesh=_MESH,
      compiler_params=params,
      scratch_types=[
          pltpu.VMEM((SEG, CHUNK), jnp.int32),
          pltpu.VMEM((SEG, CHUNK), jnp.int32),
          [pltpu.VMEM((CHUNK, dh), jnp.float32)] * NBUF,
          pltpu.VMEM_SHARED((N_PAD, dh), jnp.float32),
          pltpu.VMEM_SHARED((N_PAD, dh), jnp.float32),
          [pltpu.SemaphoreType.DMA] * NBUF,
          [pltpu.SemaphoreType.DMA] * NBUF,
      ],
  )
  def sc_aggregate(g_hbm, src2_hbm, dst2_hbm, out_hbm, src_v, dst_v, rows,
                   table_sh, acc_sh, gsem, ssem):
    c = lax.axis_index("c")
    s = lax.axis_index("s")
    rslice = pl.ds(s * ROWS_PER_TILE, ROWS_PER_TILE)

    def gather_start(j, b):
      pltpu.async_copy(table_sh.at[src_v.at[j]], rows[b], gsem[b])

    def gather_wait(j, b):
      pltpu.make_async_copy(table_sh.at[src_v.at[j]], rows[b], gsem[b]).wait()

    def scatter_start(j, b):
      pltpu.async_copy(rows[b], acc_sh.at[dst_v.at[j]], ssem[b], add=True)

    def scatter_wait(j, b):
      pltpu.make_async_copy(rows[b], acc_sh.at[dst_v.at[j]], ssem[b]).wait()

    pltpu.sync_copy(g_hbm.at[c, rslice], table_sh.at[rslice])
    _zero_fill(rows[0], CHUNK, dh)

    @pl.loop(0, ROWS_PER_TILE // CHUNK)
    def _(i):
      pltpu.sync_copy(rows[0], acc_sh.at[pl.ds(s * ROWS_PER_TILE + i * CHUNK,
                                               CHUNK)])

    plsc.subcore_barrier()

    for seg in range(NCH_T // SEG):
      row0 = s * NCH_T + seg * SEG
      pltpu.sync_copy(src2_hbm.at[pl.ds(row0, SEG)], src_v)
      pltpu.sync_copy(dst2_hbm.at[pl.ds(row0, SEG)], dst_v)
      gather_start(0, 0)
      gather_start(1, 1)

      @pl.loop(0, SEG // NBUF)
      def _(i):
        j0 = NBUF * i
        for k in range(NBUF):
          b = k
          bn = (k + 2) % NBUF
          gather_wait(j0 + k, b)
          scatter_start(j0 + k, b)
          if k < 2:
            @pl.when(i > 0)
            def _():
              scatter_wait(j0 + k - 2, bn)
            gather_start(j0 + k + 2, bn)
          else:
            scatter_wait(j0 + k - 2, bn)

            @pl.when(i < SEG // NBUF - 1)
            def _():
              gather_start(j0 + k + 2, bn)

      scatter_wait(SEG - 2, 2)
      scatter_wait(SEG - 1, 3)

    plsc.subcore_barrier()

    pltpu.sync_copy(acc_sh.at[rslice], out_hbm.at[c, rslice])

  return sc_aggregate


_sc_aggregate_h = _make_sc_aggregate(D_H // NC)
_sc_aggregate_o = _make_sc_aggregate(D_OUT // NC)


@functools.partial(
    pl.kernel,
    out_type=jax.ShapeDtypeStruct((NC, N_PAD), jnp.float32),
    mesh=_MESH,
    scratch_types=[
        pltpu.VMEM((NCHUNK, CHUNK), jnp.int32),
        pltpu.VMEM((CHUNK,), jnp.float32),
        pltpu.VMEM((ROWS_PER_TILE,), jnp.float32),
        pltpu.VMEM_SHARED((N_PAD,), jnp.float32),
    ],
)
def _sc_degree(dst2_hbm, out_hbm, dst_v, ones_v, zrow_v, acc_sh):
  c = lax.axis_index("c")
  s = lax.axis_index("s")
  w = c * NS + s

  one16 = jnp.ones((16,), jnp.float32)
  z16 = jnp.zeros((16,), jnp.float32)
  for j in range(CHUNK // 16):
    ones_v[pl.ds(j * 16, 16)] = one16

  @pl.loop(0, ROWS_PER_TILE // 16)
  def _(i):
    zrow_v[pl.ds(i * 16, 16)] = z16

  pltpu.sync_copy(zrow_v, acc_sh.at[pl.ds(s * ROWS_PER_TILE, ROWS_PER_TILE)])
  pltpu.sync_copy(dst2_hbm.at[pl.ds(w * NCHUNK, NCHUNK)], dst_v)
  plsc.subcore_barrier()

  @pl.loop(0, NCHUNK)
  def _(i):
    pltpu.sync_copy(ones_v, acc_sh.at[dst_v.at[i]], add=True)

  plsc.subcore_barrier()

  pltpu.sync_copy(acc_sh.at[pl.ds(s * ROWS_PER_TILE, ROWS_PER_TILE)],
                  out_hbm.at[c, pl.ds(s * ROWS_PER_TILE, ROWS_PER_TILE)])


def _split(h):
  return jnp.stack([h[:, :h.shape[1] // NC], h[:, h.shape[1] // NC:]])


def _cat(ref):
  return jnp.concatenate([ref[0], ref[1]], axis=-1)


def _tc1_body(x_ref, w1_ref, degp_ref, g1_ref, dinv_ref):
  deg = degp_ref[0] + degp_ref[1] + 1.0
  dinv = lax.rsqrt(jnp.maximum(deg, 1.0))
  h = jnp.dot(x_ref[...], w1_ref[...], preferred_element_type=jnp.float32)
  g1_ref[...] = _split(h * dinv)
  dinv_ref[...] = dinv


def _tc2_body(agg_ref, g1_ref, dinv_ref, b1_ref, w2_ref, g2_ref):
  agg = _cat(agg_ref) + _cat(g1_ref)
  z1 = jnp.maximum(agg * dinv_ref[...] + b1_ref[...], 0.0)
  h2 = jnp.dot(z1, w2_ref[...], preferred_element_type=jnp.float32)
  g2_ref[...] = _split(h2 * dinv_ref[...])


def _tc3_body(agg_ref, g2_ref, dinv_ref, b2_ref, out_ref):
  agg = _cat(agg_ref) + _cat(g2_ref)
  out_ref[...] = jnp.maximum(agg * dinv_ref[...] + b2_ref[...], 0.0)


_tc1 = pl.pallas_call(
    _tc1_body,
    out_shape=(jax.ShapeDtypeStruct((NC, N_PAD, D_H // NC), jnp.float32),
               jax.ShapeDtypeStruct((N_PAD, 1), jnp.float32)))

_tc2 = pl.pallas_call(
    _tc2_body,
    out_shape=jax.ShapeDtypeStruct((NC, N_PAD, D_OUT // NC), jnp.float32))

_tc3 = pl.pallas_call(
    _tc3_body,
    out_shape=jax.ShapeDtypeStruct((N_PAD, D_OUT), jnp.float32))


@jax.jit
def kernel(x, edge_index, W1, b1, W2, b2):
  src = edge_index[0].astype(jnp.int32)
  dst = edge_index[1].astype(jnp.int32)
  npad = E_PAD - E
  src_p = jnp.concatenate([src, jnp.zeros((npad,), jnp.int32)]).reshape(
      E_PAD // CHUNK, CHUNK)
  dst_p = jnp.concatenate([dst, jnp.full((npad,), N, jnp.int32)]).reshape(
      E_PAD // CHUNK, CHUNK)
  x_p = jnp.concatenate([x, jnp.zeros((N_PAD - N, D_IN), jnp.float32)])

  degp = _sc_degree(dst_p).reshape(NC, N_PAD, 1)
  g1, dinv = _tc1(x_p, W1, degp)
  agg1 = _sc_aggregate_h(g1, src_p, dst_p)
  g2 = _tc2(agg1, g1, dinv, b1.reshape(1, D_H), W2)
  agg2 = _sc_aggregate_o(g2, src_p, dst_p)
  z = _tc3(agg2, g2, dinv, b2.reshape(1, D_OUT))
  return z[:N]


# speedup vs baseline: 31.6865x; 1.0765x over previous
import functools

import jax
import jax.numpy as jnp
from jax import lax
from jax.experimental import pallas as pl
from jax.experimental.pallas import tpu as pltpu
from jax.experimental.pallas import tpu_sc as plsc

N = 10000
E = 320000
D_IN = 128
D_H = 128
D_OUT = 64

NC = 2
NS = 16
NW = NC * NS

CHUNK = 128
ROWS_E = E // CHUNK
RPT = N // NS

CPT = ROWS_E // NS
SEG = 32
NFULL = 4
TAIL = CPT - NFULL * SEG
NBUF = 4

CPW = ROWS_E // NW
N_DEG = 10240
DEG_RPT = N_DEG // NS

_MESH = plsc.VectorSubcoreMesh(core_axis_name="c", subcore_axis_name="s")


def _zero_fill(ref, nrows, width):
  z16 = jnp.zeros((16,), jnp.float32)

  @pl.loop(0, nrows)
  def _(i):
    for j in range(width // 16):
      ref[i, pl.ds(j * 16, 16)] = z16


def _make_sc_aggregate(dh):
  params = pltpu.CompilerParams(use_tc_tiling_on_sc=False)

  @functools.partial(
      pl.kernel,
      out_type=jax.ShapeDtypeStruct((NC, N, dh), jnp.float32),
      mesh=_MESH,
      compiler_params=params,
      scratch_types=[
          pltpu.VMEM((SEG, CHUNK), jnp.int32),
          pltpu.VMEM((SEG, CHUNK), jnp.int32),
          [pltpu.VMEM((CHUNK, dh), jnp.float32)] * NBUF,
          pltpu.VMEM_SHARED((N, dh), jnp.float32),
          pltpu.VMEM_SHARED((N, dh), jnp.float32),
          [pltpu.SemaphoreType.DMA] * NBUF,
          [pltpu.SemaphoreType.DMA] * NBUF,
      ],
  )
  def sc_aggregate(g_hbm, ei_hbm, out_hbm, src_v, dst_v, rows, table_sh,
                   acc_sh, gsem, ssem):
    c = lax.axis_index("c")
    s = lax.axis_index("s")
    rslice = pl.ds(s * RPT, RPT)
    base = CPT * s + jnp.minimum(s, 4)

    def gather_start(j, b):
      pltpu.async_copy(table_sh.at[src_v.at[j]], rows[b], gsem[b])

    def gather_wait(j, b):
      pltpu.make_async_copy(table_sh.at[src_v.at[j]], rows[b], gsem[b]).wait()

    def scatter_start(j, b):
      pltpu.async_copy(rows[b], acc_sh.at[dst_v.at[j]], ssem[b], add=True)

    def scatter_wait(j, b):
      pltpu.make_async_copy(rows[b], acc_sh.at[dst_v.at[j]], ssem[b]).wait()

    pltpu.sync_copy(g_hbm.at[c, rslice], table_sh.at[rslice])
    _zero_fill(rows[0], CHUNK, dh)
    for i in range(RPT // CHUNK):
      pltpu.sync_copy(rows[0], acc_sh.at[pl.ds(s * RPT + i * CHUNK, CHUNK)])
    rem = RPT % CHUNK
    pltpu.sync_copy(rows[0].at[pl.ds(0, rem)],
                    acc_sh.at[pl.ds(s * RPT + RPT - rem, rem)])

    plsc.subcore_barrier()

    def run_segment(row0, nch):
      pltpu.sync_copy(ei_hbm.at[0, pl.ds(row0, nch)],
                      src_v.at[pl.ds(0, nch)])
      pltpu.sync_copy(ei_hbm.at[1, pl.ds(row0, nch)],
                      dst_v.at[pl.ds(0, nch)])
      gather_start(0, 0)
      gather_start(1, 1)

      @pl.loop(0, nch // NBUF)
      def _(i):
        j0 = NBUF * i
        for k in range(NBUF):
          b = k
          bn = (k + 2) % NBUF
          gather_wait(j0 + k, b)
          scatter_start(j0 + k, b)
          if k < 2:
            @pl.when(i > 0)
            def _():
              scatter_wait(j0 + k - 2, bn)
            gather_start(j0 + k + 2, bn)
          else:
            scatter_wait(j0 + k - 2, bn)

            @pl.when(i < nch // NBUF - 1)
            def _():
              gather_start(j0 + k + 2, bn)

      scatter_wait(nch - 2, 2)
      scatter_wait(nch - 1, 3)

    for seg in range(NFULL):
      run_segment(base + seg * SEG, SEG)

    tail0 = base + NFULL * SEG
    run_segment(tail0, TAIL)

    @pl.when(s < 4)
    def _():
      pltpu.sync_copy(ei_hbm.at[0, pl.ds(tail0 + TAIL, 1)],
                      src_v.at[pl.ds(TAIL, 1)])
      pltpu.sync_copy(ei_hbm.at[1, pl.ds(tail0 + TAIL, 1)],
                      dst_v.at[pl.ds(TAIL, 1)])
      pltpu.async_copy(table_sh.at[src_v.at[TAIL]], rows[0], gsem[0]).wait()
      pltpu.sync_copy(rows[0], acc_sh.at[dst_v.at[TAIL]], add=True)

    plsc.subcore_barrier()

    pltpu.sync_copy(acc_sh.at[rslice], out_hbm.at[c, rslice])

  return sc_aggregate


_sc_aggregate_h = _make_sc_aggregate(D_H // NC)
_sc_aggregate_o = _make_sc_aggregate(D_OUT // NC)


@functools.partial(
    pl.kernel,
    out_type=jax.ShapeDtypeStruct((NC, N_DEG), jnp.float32),
    mesh=_MESH,
    compiler_params=pltpu.CompilerParams(use_tc_tiling_on_sc=False),
    scratch_types=[
        pltpu.VMEM((CPW + 1, CHUNK), jnp.int32),
        pltpu.VMEM((CHUNK,), jnp.float32),
        pltpu.VMEM((DEG_RPT,), jnp.float32),
        pltpu.VMEM_SHARED((N_DEG,), jnp.float32),
    ],
)
def _sc_degree(ei_hbm, out_hbm, dst_v, ones_v, zrow_v, acc_sh):
  c = lax.axis_index("c")
  s = lax.axis_index("s")
  w = c * NS + s
  base = CPW * w + jnp.minimum(w, 4)

  one16 = jnp.ones((16,), jnp.float32)
  z16 = jnp.zeros((16,), jnp.float32)
  for j in range(CHUNK // 16):
    ones_v[pl.ds(j * 16, 16)] = one16

  @pl.loop(0, DEG_RPT // 16)
  def _(i):
    zrow_v[pl.ds(i * 16, 16)] = z16

  pltpu.sync_copy(zrow_v, acc_sh.at[pl.ds(s * DEG_RPT, DEG_RPT)])
  pltpu.sync_copy(ei_hbm.at[1, pl.ds(base, CPW)], dst_v.at[pl.ds(0, CPW)])

  @pl.when(w < 4)
  def _():
    pltpu.sync_copy(ei_hbm.at[1, pl.ds(base + CPW, 1)],
                    dst_v.at[pl.ds(CPW, 1)])

  plsc.subcore_barrier()

  @pl.loop(0, CPW)
  def _(i):
    pltpu.sync_copy(ones_v, acc_sh.at[dst_v.at[i]], add=True)

  @pl.when(w < 4)
  def _():
    pltpu.sync_copy(ones_v, acc_sh.at[dst_v.at[CPW]], add=True)

  plsc.subcore_barrier()

  pltpu.sync_copy(acc_sh.at[pl.ds(s * DEG_RPT, DEG_RPT)],
                  out_hbm.at[c, pl.ds(s * DEG_RPT, DEG_RPT)])


def _split(h):
  return jnp.stack([h[:, :h.shape[1] // NC], h[:, h.shape[1] // NC:]])


def _cat(ref):
  return jnp.concatenate([ref[0], ref[1]], axis=-1)


def _tc1_body(x_ref, w1_ref, degp_ref, g1_ref, dinv_ref):
  deg = degp_ref[0] + degp_ref[1] + 1.0
  dinv = lax.rsqrt(jnp.maximum(deg, 1.0))
  h = jnp.dot(x_ref[...], w1_ref[...], preferred_element_type=jnp.float32)
  g1_ref[...] = _split(h * dinv)
  dinv_ref[...] = dinv


def _tc2_body(agg_ref, g1_ref, dinv_ref, b1_ref, w2_ref, g2_ref):
  agg = _cat(agg_ref) + _cat(g1_ref)
  z1 = jnp.maximum(agg * dinv_ref[...] + b1_ref[...], 0.0)
  h2 = jnp.dot(z1, w2_ref[...], preferred_element_type=jnp.float32)
  g2_ref[...] = _split(h2 * dinv_ref[...])


def _tc3_body(agg_ref, g2_ref, dinv_ref, b2_ref, out_ref):
  agg = _cat(agg_ref) + _cat(g2_ref)
  out_ref[...] = jnp.maximum(agg * dinv_ref[...] + b2_ref[...], 0.0)


_tc1 = pl.pallas_call(
    _tc1_body,
    out_shape=(jax.ShapeDtypeStruct((NC, N, D_H // NC), jnp.float32),
               jax.ShapeDtypeStruct((N, 1), jnp.float32)))

_tc2 = pl.pallas_call(
    _tc2_body,
    out_shape=jax.ShapeDtypeStruct((NC, N, D_OUT // NC), jnp.float32))

_tc3 = pl.pallas_call(
    _tc3_body,
    out_shape=jax.ShapeDtypeStruct((N, D_OUT), jnp.float32))


@jax.jit
def kernel(x, edge_index, W1, b1, W2, b2):
  ei = edge_index.astype(jnp.int32).reshape(2, ROWS_E, CHUNK)
  degp = _sc_degree(ei)[:, :N].reshape(NC, N, 1)
  g1, dinv = _tc1(x, W1, degp)
  agg1 = _sc_aggregate_h(g1, ei)
  g2 = _tc2(agg1, g1, dinv, b1.reshape(1, D_H), W2)
  agg2 = _sc_aggregate_o(g2, ei)
  return _tc3(agg2, g2, dinv, b2.reshape(1, D_OUT))


# speedup vs baseline: 35.4948x; 1.1202x over previous
import functools

import jax
import jax.numpy as jnp
from jax import lax
from jax.experimental import pallas as pl
from jax.experimental.pallas import tpu as pltpu
from jax.experimental.pallas import tpu_sc as plsc

N = 10000
E = 320000
D_IN = 128
D_H = 128
D_OUT = 64

NC = 2
NS = 16
NW = NC * NS

CHUNK = 128
ROWS_E = E // CHUNK
RPT = N // NS

CPT = ROWS_E // NS
SEG = 32
NFULL = 4
TAIL = CPT - NFULL * SEG
NBUF = 4

CPW = ROWS_E // NW
N_DEG = 10240
DEG_RPT = N_DEG // NS

_MESH = plsc.VectorSubcoreMesh(core_axis_name="c", subcore_axis_name="s")


def _zero_fill(ref, nrows, width):
  z16 = jnp.zeros((16,), jnp.float32)

  @pl.loop(0, nrows)
  def _(i):
    for j in range(width // 16):
      ref[i, pl.ds(j * 16, 16)] = z16


def _make_sc_aggregate(dh):
  params = pltpu.CompilerParams(use_tc_tiling_on_sc=False)

  @functools.partial(
      pl.kernel,
      out_type=jax.ShapeDtypeStruct((N, NC * dh), jnp.float32),
      mesh=_MESH,
      compiler_params=params,
      scratch_types=[
          pltpu.VMEM((SEG, CHUNK), jnp.int32),
          pltpu.VMEM((SEG, CHUNK), jnp.int32),
          [pltpu.VMEM((CHUNK, dh), jnp.float32)] * NBUF,
          pltpu.VMEM_SHARED((N, dh), jnp.float32),
          pltpu.VMEM_SHARED((N, dh), jnp.float32),
          [pltpu.SemaphoreType.DMA] * NBUF,
          [pltpu.SemaphoreType.DMA] * NBUF,
      ],
  )
  def sc_aggregate(g_hbm, ei_hbm, out_hbm, src_v, dst_v, rows, table_sh,
                   acc_sh, gsem, ssem):
    c = lax.axis_index("c")
    s = lax.axis_index("s")
    rslice = pl.ds(s * RPT, RPT)
    base = CPT * s + jnp.minimum(s, 4)

    def gather_start(j, b):
      pltpu.async_copy(table_sh.at[src_v.at[j]], rows[b], gsem[b])

    def gather_wait(j, b):
      pltpu.make_async_copy(table_sh.at[src_v.at[j]], rows[b], gsem[b]).wait()

    def scatter_start(j, b):
      pltpu.async_copy(rows[b], acc_sh.at[dst_v.at[j]], ssem[b], add=True)

    def scatter_wait(j, b):
      pltpu.make_async_copy(rows[b], acc_sh.at[dst_v.at[j]], ssem[b]).wait()

    pltpu.sync_copy(g_hbm.at[rslice, pl.ds(c * dh, dh)], table_sh.at[rslice])
    _zero_fill(rows[0], CHUNK, dh)
    for i in range(RPT // CHUNK):
      pltpu.sync_copy(rows[0], acc_sh.at[pl.ds(s * RPT + i * CHUNK, CHUNK)])
    rem = RPT % CHUNK
    pltpu.sync_copy(rows[0].at[pl.ds(0, rem)],
                    acc_sh.at[pl.ds(s * RPT + RPT - rem, rem)])

    plsc.subcore_barrier()

    def run_segment(row0, nch):
      pltpu.sync_copy(ei_hbm.at[0, pl.ds(row0, nch)],
                      src_v.at[pl.ds(0, nch)])
      pltpu.sync_copy(ei_hbm.at[1, pl.ds(row0, nch)],
                      dst_v.at[pl.ds(0, nch)])
      gather_start(0, 0)
      gather_start(1, 1)

      @pl.loop(0, nch // NBUF)
      def _(i):
        j0 = NBUF * i
        for k in range(NBUF):
          b = k
          bn = (k + 2) % NBUF
          gather_wait(j0 + k, b)
          scatter_start(j0 + k, b)
          if k < 2:
            @pl.when(i > 0)
            def _():
              scatter_wait(j0 + k - 2, bn)
            gather_start(j0 + k + 2, bn)
          else:
            scatter_wait(j0 + k - 2, bn)

            @pl.when(i < nch // NBUF - 1)
            def _():
              gather_start(j0 + k + 2, bn)

      scatter_wait(nch - 2, 2)
      scatter_wait(nch - 1, 3)

    for seg in range(NFULL):
      run_segment(base + seg * SEG, SEG)

    tail0 = base + NFULL * SEG
    run_segment(tail0, TAIL)

    @pl.when(s < 4)
    def _():
      pltpu.sync_copy(ei_hbm.at[0, pl.ds(tail0 + TAIL, 1)],
                      src_v.at[pl.ds(TAIL, 1)])
      pltpu.sync_copy(ei_hbm.at[1, pl.ds(tail0 + TAIL, 1)],
                      dst_v.at[pl.ds(TAIL, 1)])
      pltpu.async_copy(table_sh.at[src_v.at[TAIL]], rows[0], gsem[0]).wait()
      pltpu.sync_copy(rows[0], acc_sh.at[dst_v.at[TAIL]], add=True)

    plsc.subcore_barrier()

    pltpu.sync_copy(acc_sh.at[rslice], out_hbm.at[rslice, pl.ds(c * dh, dh)])

  return sc_aggregate


_sc_aggregate_h = _make_sc_aggregate(D_H // NC)
_sc_aggregate_o = _make_sc_aggregate(D_OUT // NC)


@functools.partial(
    pl.kernel,
    out_type=jax.ShapeDtypeStruct((NC, N_DEG), jnp.float32),
    mesh=_MESH,
    compiler_params=pltpu.CompilerParams(use_tc_tiling_on_sc=False),
    scratch_types=[
        pltpu.VMEM((CPW + 1, CHUNK), jnp.int32),
        pltpu.VMEM((CHUNK,), jnp.float32),
        pltpu.VMEM((DEG_RPT,), jnp.float32),
        pltpu.VMEM_SHARED((N_DEG,), jnp.float32),
    ],
)
def _sc_degree(ei_hbm, out_hbm, dst_v, ones_v, zrow_v, acc_sh):
  c = lax.axis_index("c")
  s = lax.axis_index("s")
  w = c * NS + s
  base = CPW * w + jnp.minimum(w, 4)

  one16 = jnp.ones((16,), jnp.float32)
  z16 = jnp.zeros((16,), jnp.float32)
  for j in range(CHUNK // 16):
    ones_v[pl.ds(j * 16, 16)] = one16

  @pl.loop(0, DEG_RPT // 16)
  def _(i):
    zrow_v[pl.ds(i * 16, 16)] = z16

  pltpu.sync_copy(zrow_v, acc_sh.at[pl.ds(s * DEG_RPT, DEG_RPT)])
  pltpu.sync_copy(ei_hbm.at[1, pl.ds(base, CPW)], dst_v.at[pl.ds(0, CPW)])

  @pl.when(w < 4)
  def _():
    pltpu.sync_copy(ei_hbm.at[1, pl.ds(base + CPW, 1)],
                    dst_v.at[pl.ds(CPW, 1)])

  plsc.subcore_barrier()

  @pl.loop(0, CPW)
  def _(i):
    pltpu.sync_copy(ones_v, acc_sh.at[dst_v.at[i]], add=True)

  @pl.when(w < 4)
  def _():
    pltpu.sync_copy(ones_v, acc_sh.at[dst_v.at[CPW]], add=True)

  plsc.subcore_barrier()

  pltpu.sync_copy(acc_sh.at[pl.ds(s * DEG_RPT, DEG_RPT)],
                  out_hbm.at[c, pl.ds(s * DEG_RPT, DEG_RPT)])


def _tc1_body(x_ref, w1_ref, degp_ref, g1_ref, dinv_ref):
  deg = degp_ref[0] + degp_ref[1] + 1.0
  dinv = lax.rsqrt(jnp.maximum(deg, 1.0))
  h = jnp.dot(x_ref[...], w1_ref[...], preferred_element_type=jnp.float32)
  g1_ref[...] = h * dinv
  dinv_ref[...] = dinv


def _tc2_body(agg_ref, g1_ref, dinv_ref, b1_ref, w2_ref, g2_ref):
  agg = agg_ref[...] + g1_ref[...]
  z1 = jnp.maximum(agg * dinv_ref[...] + b1_ref[...], 0.0)
  h2 = jnp.dot(z1, w2_ref[...], preferred_element_type=jnp.float32)
  g2_ref[...] = h2 * dinv_ref[...]


def _tc3_body(agg_ref, g2_ref, dinv_ref, b2_ref, out_ref):
  agg = agg_ref[...] + g2_ref[...]
  out_ref[...] = jnp.maximum(agg * dinv_ref[...] + b2_ref[...], 0.0)


_tc1 = pl.pallas_call(
    _tc1_body,
    out_shape=(jax.ShapeDtypeStruct((N, D_H), jnp.float32),
               jax.ShapeDtypeStruct((N, 1), jnp.float32)))

_tc2 = pl.pallas_call(
    _tc2_body,
    out_shape=jax.ShapeDtypeStruct((N, D_OUT), jnp.float32))

_tc3 = pl.pallas_call(
    _tc3_body,
    out_shape=jax.ShapeDtypeStruct((N, D_OUT), jnp.float32))


@jax.jit
def kernel(x, edge_index, W1, b1, W2, b2):
  ei = edge_index.astype(jnp.int32).reshape(2, ROWS_E, CHUNK)
  degp = _sc_degree(ei)[:, :N].reshape(NC, N, 1)
  g1, dinv = _tc1(x, W1, degp)
  agg1 = _sc_aggregate_h(g1, ei)
  g2 = _tc2(agg1, g1, dinv, b1.reshape(1, D_H), W2)
  agg2 = _sc_aggregate_o(g2, ei)
  return _tc3(agg2, g2, dinv, b2.reshape(1, D_OUT))


# speedup vs baseline: 38.6784x; 1.0897x over previous
import functools

import jax
import jax.numpy as jnp
from jax import lax
from jax.experimental import pallas as pl
from jax.experimental.pallas import tpu as pltpu
from jax.experimental.pallas import tpu_sc as plsc

N = 10000
E = 320000
D_IN = 128
D_H = 128
D_OUT = 64

NC = 2
NS = 16
NW = NC * NS

CHUNK = 128
ROWS_E = E // CHUNK
RPT = N // NS

CPT = ROWS_E // NS
SEG = 32
NFULL = 4
TAIL = CPT - NFULL * SEG
NBUF = 4

CPW = ROWS_E // NW
N_DEG = 10240
DEG_RPT = N_DEG // NS

_MESH = plsc.VectorSubcoreMesh(core_axis_name="c", subcore_axis_name="s")


def _zero_fill(ref, nrows, width):
  z16 = jnp.zeros((16,), jnp.float32)

  @pl.loop(0, nrows)
  def _(i):
    for j in range(width // 16):
      ref[i, pl.ds(j * 16, 16)] = z16


def _make_sc_aggregate(dh, wide):
  params = pltpu.CompilerParams(use_tc_tiling_on_sc=False)

  @functools.partial(
      pl.kernel,
      out_type=jax.ShapeDtypeStruct((N, wide), jnp.float32),
      mesh=_MESH,
      compiler_params=params,
      scratch_types=[
          pltpu.VMEM((SEG, CHUNK), jnp.int32),
          pltpu.VMEM((SEG, CHUNK), jnp.int32),
          [pltpu.VMEM((CHUNK, dh), jnp.float32)] * NBUF,
          pltpu.VMEM_SHARED((N, dh), jnp.float32),
          pltpu.VMEM_SHARED((N, dh), jnp.float32),
          [pltpu.SemaphoreType.DMA] * NBUF,
          [pltpu.SemaphoreType.DMA] * NBUF,
      ],
  )
  def sc_aggregate(g_hbm, ei_hbm, out_hbm, src_v, dst_v, rows, table_sh,
                   acc_sh, gsem, ssem):
    c = lax.axis_index("c")
    s = lax.axis_index("s")
    rslice = pl.ds(s * RPT, RPT)
    base = CPT * s + jnp.minimum(s, 4)

    def gather_start(j, b):
      pltpu.async_copy(table_sh.at[src_v.at[j]], rows[b], gsem[b])

    def gather_wait(j, b):
      pltpu.make_async_copy(table_sh.at[src_v.at[j]], rows[b], gsem[b]).wait()

    def scatter_start(j, b):
      pltpu.async_copy(rows[b], acc_sh.at[dst_v.at[j]], ssem[b], add=True)

    def scatter_wait(j, b):
      pltpu.make_async_copy(rows[b], acc_sh.at[dst_v.at[j]], ssem[b]).wait()

    pltpu.sync_copy(g_hbm.at[rslice, pl.ds(c * dh, dh)], table_sh.at[rslice])
    _zero_fill(rows[0], CHUNK, dh)
    for i in range(RPT // CHUNK):
      pltpu.sync_copy(rows[0], acc_sh.at[pl.ds(s * RPT + i * CHUNK, CHUNK)])
    rem = RPT % CHUNK
    pltpu.sync_copy(rows[0].at[pl.ds(0, rem)],
                    acc_sh.at[pl.ds(s * RPT + RPT - rem, rem)])

    plsc.subcore_barrier()

    def run_segment(row0, nch):
      pltpu.sync_copy(ei_hbm.at[0, pl.ds(row0, nch)],
                      src_v.at[pl.ds(0, nch)])
      pltpu.sync_copy(ei_hbm.at[1, pl.ds(row0, nch)],
                      dst_v.at[pl.ds(0, nch)])
      gather_start(0, 0)
      gather_start(1, 1)

      @pl.loop(0, nch // NBUF)
      def _(i):
        j0 = NBUF * i
        for k in range(NBUF):
          b = k
          bn = (k + 2) % NBUF
          gather_wait(j0 + k, b)
          scatter_start(j0 + k, b)
          if k < 2:
            @pl.when(i > 0)
            def _():
              scatter_wait(j0 + k - 2, bn)
            gather_start(j0 + k + 2, bn)
          else:
            scatter_wait(j0 + k - 2, bn)

            @pl.when(i < nch // NBUF - 1)
            def _():
              gather_start(j0 + k + 2, bn)

      scatter_wait(nch - 2, 2)
      scatter_wait(nch - 1, 3)

    for seg in range(NFULL):
      run_segment(base + seg * SEG, SEG)

    tail0 = base + NFULL * SEG
    run_segment(tail0, TAIL)

    @pl.when(s < 4)
    def _():
      pltpu.sync_copy(ei_hbm.at[0, pl.ds(tail0 + TAIL, 1)],
                      src_v.at[pl.ds(TAIL, 1)])
      pltpu.sync_copy(ei_hbm.at[1, pl.ds(tail0 + TAIL, 1)],
                      dst_v.at[pl.ds(TAIL, 1)])
      pltpu.async_copy(table_sh.at[src_v.at[TAIL]], rows[0], gsem[0]).wait()
      pltpu.sync_copy(rows[0], acc_sh.at[dst_v.at[TAIL]], add=True)

    plsc.subcore_barrier()

    pltpu.sync_copy(acc_sh.at[rslice], out_hbm.at[rslice, pl.ds(c * dh, dh)])

  return sc_aggregate


_sc_aggregate_h = _make_sc_aggregate(D_H // NC, D_H)
_sc_aggregate_o = _make_sc_aggregate(D_OUT // NC, D_H)


@functools.partial(
    pl.kernel,
    out_type=jax.ShapeDtypeStruct((NC, N_DEG), jnp.float32),
    mesh=_MESH,
    compiler_params=pltpu.CompilerParams(use_tc_tiling_on_sc=False),
    scratch_types=[
        pltpu.VMEM((CPW + 1, CHUNK), jnp.int32),
        pltpu.VMEM((CHUNK,), jnp.float32),
        pltpu.VMEM((DEG_RPT,), jnp.float32),
        pltpu.VMEM_SHARED((N_DEG,), jnp.float32),
    ],
)
def _sc_degree(ei_hbm, out_hbm, dst_v, ones_v, zrow_v, acc_sh):
  c = lax.axis_index("c")
  s = lax.axis_index("s")
  w = c * NS + s
  base = CPW * w + jnp.minimum(w, 4)

  one16 = jnp.ones((16,), jnp.float32)
  z16 = jnp.zeros((16,), jnp.float32)
  for j in range(CHUNK // 16):
    ones_v[pl.ds(j * 16, 16)] = one16

  @pl.loop(0, DEG_RPT // 16)
  def _(i):
    zrow_v[pl.ds(i * 16, 16)] = z16

  pltpu.sync_copy(zrow_v, acc_sh.at[pl.ds(s * DEG_RPT, DEG_RPT)])
  pltpu.sync_copy(ei_hbm.at[1, pl.ds(base, CPW)], dst_v.at[pl.ds(0, CPW)])

  @pl.when(w < 4)
  def _():
    pltpu.sync_copy(ei_hbm.at[1, pl.ds(base + CPW, 1)],
                    dst_v.at[pl.ds(CPW, 1)])

  plsc.subcore_barrier()

  @pl.loop(0, CPW)
  def _(i):
    pltpu.sync_copy(ones_v, acc_sh.at[dst_v.at[i]], add=True)

  @pl.when(w < 4)
  def _():
    pltpu.sync_copy(ones_v, acc_sh.at[dst_v.at[CPW]], add=True)

  plsc.subcore_barrier()

  pltpu.sync_copy(acc_sh.at[pl.ds(s * DEG_RPT, DEG_RPT)],
                  out_hbm.at[c, pl.ds(s * DEG_RPT, DEG_RPT)])


def _tc1_body(x_ref, w1_ref, degp_ref, g1_ref, dinv_ref):
  deg = degp_ref[0, :N] + degp_ref[1, :N] + 1.0
  dinv = lax.rsqrt(jnp.maximum(deg, 1.0))
  h = jnp.dot(x_ref[...], w1_ref[...], preferred_element_type=jnp.float32)
  g1_ref[...] = h * dinv[:, None]
  dinv_ref[...] = dinv


def _tc2_body(agg_ref, g1_ref, dinv_ref, b1_ref, w2_ref, g2_ref):
  dinv = dinv_ref[...][:, None]
  agg = agg_ref[...] + g1_ref[...]
  z1 = jnp.maximum(agg * dinv + b1_ref[...], 0.0)
  h2 = jnp.dot(z1, w2_ref[...], preferred_element_type=jnp.float32)
  g2 = h2 * dinv
  g2_ref[...] = jnp.concatenate([g2, jnp.zeros((N, D_H - D_OUT), g2.dtype)],
                                axis=-1)


def _tc3_body(agg_ref, g2_ref, dinv_ref, b2_ref, out_ref):
  agg = agg_ref[:, :D_OUT] + g2_ref[:, :D_OUT]
  out_ref[...] = jnp.maximum(agg * dinv_ref[...][:, None] + b2_ref[...], 0.0)


_tc1 = pl.pallas_call(
    _tc1_body,
    out_shape=(jax.ShapeDtypeStruct((N, D_H), jnp.float32),
               jax.ShapeDtypeStruct((N,), jnp.float32)))

_tc2 = pl.pallas_call(
    _tc2_body,
    out_shape=jax.ShapeDtypeStruct((N, D_H), jnp.float32))

_tc3 = pl.pallas_call(
    _tc3_body,
    out_shape=jax.ShapeDtypeStruct((N, D_OUT), jnp.float32))


@jax.jit
def kernel(x, edge_index, W1, b1, W2, b2):
  ei = edge_index.astype(jnp.int32).reshape(2, ROWS_E, CHUNK)
  degp = _sc_degree(ei)
  g1, dinv = _tc1(x, W1, degp)
  agg1 = _sc_aggregate_h(g1, ei)
  g2 = _tc2(agg1, g1, dinv, b1.reshape(1, D_H), W2)
  agg2 = _sc_aggregate_o(g2, ei)
  return _tc3(agg2, g2, dinv, b2.reshape(1, D_OUT))


# speedup vs baseline: 39.8433x; 1.0301x over previous
import functools

import jax
import jax.numpy as jnp
from jax import lax
from jax.experimental import pallas as pl
from jax.experimental.pallas import tpu as pltpu
from jax.experimental.pallas import tpu_sc as plsc

N = 10000
E = 320000
D_IN = 128
D_H = 128
D_OUT = 64

NC = 2
NS = 16
NW = NC * NS

CHUNK = 128
ROWS_E = E // CHUNK
RPT = N // NS

CPT = ROWS_E // NS
SEG = 52
NFULL = 2
TAIL = CPT - NFULL * SEG
NBUF = 4

CPW = ROWS_E // NW
N_DEG = 10240
DEG_RPT = N_DEG // NS

_MESH = plsc.VectorSubcoreMesh(core_axis_name="c", subcore_axis_name="s")


def _zero_fill(ref, nrows, width):
  z16 = jnp.zeros((16,), jnp.float32)

  @pl.loop(0, nrows)
  def _(i):
    for j in range(width // 16):
      ref[i, pl.ds(j * 16, 16)] = z16


def _make_sc_aggregate(dh, wide):
  params = pltpu.CompilerParams(use_tc_tiling_on_sc=False)

  @functools.partial(
      pl.kernel,
      out_type=jax.ShapeDtypeStruct((N, wide), jnp.float32),
      mesh=_MESH,
      compiler_params=params,
      scratch_types=[
          pltpu.VMEM((SEG + 1, CHUNK), jnp.int32),
          pltpu.VMEM((SEG + 1, CHUNK), jnp.int32),
          [pltpu.VMEM((CHUNK, dh), jnp.float32)] * NBUF,
          pltpu.VMEM_SHARED((N, dh), jnp.float32),
          pltpu.VMEM_SHARED((N, dh), jnp.float32),
          [pltpu.SemaphoreType.DMA] * NBUF,
          [pltpu.SemaphoreType.DMA] * NBUF,
      ],
  )
  def sc_aggregate(g_hbm, ei_hbm, out_hbm, src_v, dst_v, rows, table_sh,
                   acc_sh, gsem, ssem):
    c = lax.axis_index("c")
    s = lax.axis_index("s")
    rslice = pl.ds(s * RPT, RPT)
    base = CPT * s + jnp.minimum(s, 4)

    def gather_start(j, b):
      pltpu.async_copy(table_sh.at[src_v.at[j]], rows[b], gsem[b])

    def gather_wait(j, b):
      pltpu.make_async_copy(table_sh.at[src_v.at[j]], rows[b], gsem[b]).wait()

    def scatter_start(j, b):
      pltpu.async_copy(rows[b], acc_sh.at[dst_v.at[j]], ssem[b], add=True)

    def scatter_wait(j, b):
      pltpu.make_async_copy(rows[b], acc_sh.at[dst_v.at[j]], ssem[b]).wait()

    pltpu.sync_copy(g_hbm.at[rslice, pl.ds(c * dh, dh)], table_sh.at[rslice])
    _zero_fill(rows[0], CHUNK, dh)
    for i in range(RPT // CHUNK):
      pltpu.sync_copy(rows[0], acc_sh.at[pl.ds(s * RPT + i * CHUNK, CHUNK)])
    rem = RPT % CHUNK
    pltpu.sync_copy(rows[0].at[pl.ds(0, rem)],
                    acc_sh.at[pl.ds(s * RPT + RPT - rem, rem)])

    plsc.subcore_barrier()

    def run_segment(row0, nch):
      pltpu.sync_copy(ei_hbm.at[0, pl.ds(row0, nch)],
                      src_v.at[pl.ds(0, nch)])
      pltpu.sync_copy(ei_hbm.at[1, pl.ds(row0, nch)],
                      dst_v.at[pl.ds(0, nch)])
      gather_start(0, 0)
      gather_start(1, 1)

      @pl.loop(0, nch // NBUF)
      def _(i):
        j0 = NBUF * i
        for k in range(NBUF):
          b = k
          bn = (k + 2) % NBUF
          gather_wait(j0 + k, b)
          scatter_start(j0 + k, b)
          if k < 2:
            @pl.when(i > 0)
            def _():
              scatter_wait(j0 + k - 2, bn)
            gather_start(j0 + k + 2, bn)
          else:
            scatter_wait(j0 + k - 2, bn)

            @pl.when(i < nch // NBUF - 1)
            def _():
              gather_start(j0 + k + 2, bn)

      scatter_wait(nch - 2, 2)
      scatter_wait(nch - 1, 3)

    for seg in range(NFULL):
      run_segment(base + seg * SEG, SEG)

    tail0 = base + NFULL * SEG
    run_segment(tail0, TAIL)

    @pl.when(s < 4)
    def _():
      pltpu.sync_copy(ei_hbm.at[0, pl.ds(tail0 + TAIL, 1)],
                      src_v.at[pl.ds(TAIL, 1)])
      pltpu.sync_copy(ei_hbm.at[1, pl.ds(tail0 + TAIL, 1)],
                      dst_v.at[pl.ds(TAIL, 1)])
      pltpu.async_copy(table_sh.at[src_v.at[TAIL]], rows[0], gsem[0]).wait()
      pltpu.sync_copy(rows[0], acc_sh.at[dst_v.at[TAIL]], add=True)

    plsc.subcore_barrier()

    pltpu.sync_copy(acc_sh.at[rslice], out_hbm.at[rslice, pl.ds(c * dh, dh)])

  return sc_aggregate


_sc_aggregate_h = _make_sc_aggregate(D_H // NC, D_H)
_sc_aggregate_o = _make_sc_aggregate(D_OUT // NC, D_H)


@functools.partial(
    pl.kernel,
    out_type=jax.ShapeDtypeStruct((NC, N_DEG), jnp.float32),
    mesh=_MESH,
    compiler_params=pltpu.CompilerParams(use_tc_tiling_on_sc=False),
    scratch_types=[
        pltpu.VMEM((CPW + 1, CHUNK), jnp.int32),
        pltpu.VMEM((CHUNK,), jnp.float32),
        pltpu.VMEM((DEG_RPT,), jnp.float32),
        pltpu.VMEM_SHARED((N_DEG,), jnp.float32),
    ],
)
def _sc_degree(ei_hbm, out_hbm, dst_v, ones_v, zrow_v, acc_sh):
  c = lax.axis_index("c")
  s = lax.axis_index("s")
  w = c * NS + s
  base = CPW * w + jnp.minimum(w, 4)

  one16 = jnp.ones((16,), jnp.float32)
  z16 = jnp.zeros((16,), jnp.float32)
  for j in range(CHUNK // 16):
    ones_v[pl.ds(j * 16, 16)] = one16

  @pl.loop(0, DEG_RPT // 16)
  def _(i):
    zrow_v[pl.ds(i * 16, 16)] = z16

  pltpu.sync_copy(zrow_v, acc_sh.at[pl.ds(s * DEG_RPT, DEG_RPT)])
  pltpu.sync_copy(ei_hbm.at[1, pl.ds(base, CPW)], dst_v.at[pl.ds(0, CPW)])

  @pl.when(w < 4)
  def _():
    pltpu.sync_copy(ei_hbm.at[1, pl.ds(base + CPW, 1)],
                    dst_v.at[pl.ds(CPW, 1)])

  plsc.subcore_barrier()

  @pl.loop(0, CPW)
  def _(i):
    pltpu.sync_copy(ones_v, acc_sh.at[dst_v.at[i]], add=True)

  @pl.when(w < 4)
  def _():
    pltpu.sync_copy(ones_v, acc_sh.at[dst_v.at[CPW]], add=True)

  plsc.subcore_barrier()

  pltpu.sync_copy(acc_sh.at[pl.ds(s * DEG_RPT, DEG_RPT)],
                  out_hbm.at[c, pl.ds(s * DEG_RPT, DEG_RPT)])


def _tc1_body(x_ref, w1_ref, degp_ref, g1_ref, dinv_ref):
  deg = degp_ref[0, :N] + degp_ref[1, :N] + 1.0
  dinv = lax.rsqrt(jnp.maximum(deg, 1.0))
  h = jnp.dot(x_ref[...], w1_ref[...], preferred_element_type=jnp.float32)
  g1_ref[...] = h * dinv[:, None]
  dinv_ref[...] = dinv


def _tc2_body(agg_ref, g1_ref, dinv_ref, b1_ref, w2_ref, g2_ref):
  dinv = dinv_ref[...][:, None]
  agg = agg_ref[...] + g1_ref[...]
  z1 = jnp.maximum(agg * dinv + b1_ref[...], 0.0)
  h2 = jnp.dot(z1, w2_ref[...], preferred_element_type=jnp.float32)
  g2 = h2 * dinv
  g2_ref[...] = jnp.concatenate([g2, jnp.zeros((N, D_H - D_OUT), g2.dtype)],
                                axis=-1)


def _tc3_body(agg_ref, g2_ref, dinv_ref, b2_ref, out_ref):
  agg = agg_ref[:, :D_OUT] + g2_ref[:, :D_OUT]
  out_ref[...] = jnp.maximum(agg * dinv_ref[...][:, None] + b2_ref[...], 0.0)


_tc1 = pl.pallas_call(
    _tc1_body,
    out_shape=(jax.ShapeDtypeStruct((N, D_H), jnp.float32),
               jax.ShapeDtypeStruct((N,), jnp.float32)))

_tc2 = pl.pallas_call(
    _tc2_body,
    out_shape=jax.ShapeDtypeStruct((N, D_H), jnp.float32))

_tc3 = pl.pallas_call(
    _tc3_body,
    out_shape=jax.ShapeDtypeStruct((N, D_OUT), jnp.float32))


@jax.jit
def kernel(x, edge_index, W1, b1, W2, b2):
  ei = edge_index.astype(jnp.int32).reshape(2, ROWS_E, CHUNK)
  degp = _sc_degree(ei)
  g1, dinv = _tc1(x, W1, degp)
  agg1 = _sc_aggregate_h(g1, ei)
  g2 = _tc2(agg1, g1, dinv, b1.reshape(1, D_H), W2)
  agg2 = _sc_aggregate_o(g2, ei)
  return _tc3(agg2, g2, dinv, b2.reshape(1, D_OUT))
